# baseline (speedup 1.0000x reference)
import functools

import jax
import jax.numpy as jnp
from jax import lax
from jax.experimental import pallas as pl
from jax.experimental.pallas import tpu as pltpu
from jax.experimental.pallas import tpu_sc as plsc

OUT = 7
SR = 2
NBIN = OUT * OUT
NTERM = SR * SR * 4
P = NBIN * NTERM
C = 128
NLANE = 16
NC8 = C // NLANE
AUXF = 1792
WOFF = 896
C0ROWS = NTERM * OUT
GROWS = 2 * C0ROWS
NGRP = 3
SIZES = (128, 64, 32, 16)
BASES = (0.0, 16384.0, 20480.0, 21504.0)
NROWS = 21760
OBOX = C * NBIN


def _prep_body(b1_ref, b2_ref, aux_ref):
    b1 = b1_ref[...]
    b2 = b2_ref[...]
    n = b1.shape[0]

    ux1 = jnp.minimum(b1[:, 0:1], b2[:, 0:1])
    uy1 = jnp.minimum(b1[:, 1:2], b2[:, 1:2])
    ux2 = jnp.maximum(b1[:, 2:3], b2[:, 2:3])
    uy2 = jnp.maximum(b1[:, 3:4], b2[:, 3:4])

    s1 = jnp.sqrt((b1[:, 2:3] - b1[:, 0:1]) * (b1[:, 3:4] - b1[:, 1:2]))
    s2 = jnp.sqrt((b2[:, 2:3] - b2[:, 0:1]) * (b2[:, 3:4] - b2[:, 1:2]))
    s = jnp.minimum(s1, s2)
    lvl = jnp.clip(jnp.floor(4.0 + jnp.log2(s / 224.0 + 1e-6)), 2.0, 5.0) - 2.0

    scale = jnp.where(lvl == 0.0, 0.25,
            jnp.where(lvl == 1.0, 0.125,
            jnp.where(lvl == 2.0, 0.0625, 0.03125))).astype(jnp.float32)
    hf = 512.0 * scale
    base = jnp.where(lvl == 0.0, BASES[0],
           jnp.where(lvl == 1.0, BASES[1],
           jnp.where(lvl == 2.0, BASES[2], BASES[3]))).astype(jnp.float32)

    x1s = ux1 * scale
    y1s = uy1 * scale
    x2s = ux2 * scale
    y2s = uy2 * scale
    rw = jnp.maximum(x2s - x1s, 1.0)
    rh = jnp.maximum(y2s - y1s, 1.0)
    bw = rw / OUT
    bh = rh / OUT

    ff = lax.broadcasted_iota(jnp.int32, (n, AUXF), 1)
    is_idx = ff < P
    is_w = (ff >= WOFF) & (ff < WOFF + P)
    pf = jnp.clip(ff - jnp.where(is_idx, 0, WOFF), 0, P - 1).astype(
        jnp.float32)

    binf = jnp.floor(pf * 0.0625)
    kf = pf - 16.0 * binf
    i_f = jnp.floor(binf / 7.0)
    j_f = binf - 7.0 * i_f
    spf = jnp.floor(kf * 0.25)
    crf = kf - 4.0 * spf
    s_f = jnp.floor(spf * 0.5)
    t_f = spf - 2.0 * s_f
    a_f = jnp.floor(crf * 0.5)
    b_f = crf - 2.0 * a_f

    gy = y1s + (i_f + (s_f + 0.5) * 0.5) * bh
    gx = x1s + (j_f + (t_f + 0.5) * 0.5) * bw
    valid = ((gy > -1.0) & (gy < hf) & (gx > -1.0) & (gx < hf))
    y = jnp.clip(gy, 0.0, hf - 1.0)
    x = jnp.clip(gx, 0.0, hf - 1.0)
    y0f = jnp.floor(y)
    x0f = jnp.floor(x)
    y1f = jnp.minimum(y0f + 1.0, hf - 1.0)
    x1f = jnp.minimum(x0f + 1.0, hf - 1.0)
    ly = y - y0f
    lx = x - x0f
    wy = jnp.where(a_f == 0.0, 1.0 - ly, ly)
    wx = jnp.where(b_f == 0.0, 1.0 - lx, lx)
    wgt = wy * wx * 0.25 * valid.astype(jnp.float32)
    ysel = jnp.where(a_f == 0.0, y0f, y1f)
    xsel = jnp.where(b_f == 0.0, x0f, x1f)
    rowf = base + ysel * hf + xsel

    aux_ref[...] = jnp.where(
        is_idx, rowf.astype(jnp.int32),
        jnp.where(is_w, jax.lax.bitcast_convert_type(wgt, jnp.int32), 0))


def _prep(boxes1, boxes2):
    m = boxes1.shape[0]
    chunk = 200 if m % 200 == 0 else m
    grid = m // chunk
    return pl.pallas_call(
        _prep_body,
        grid=(grid,),
        in_specs=[
            pl.BlockSpec((chunk, 4), lambda g: (g, 0)),
            pl.BlockSpec((chunk, 4), lambda g: (g, 0)),
        ],
        out_specs=pl.BlockSpec((chunk, AUXF), lambda g: (g, 0)),
        out_shape=jax.ShapeDtypeStruct((m, AUXF), jnp.int32),
    )(boxes1, boxes2)


def _tpose_body(x_ref, o_ref):
    o_ref[...] = x_ref[...].T


def _tpose(x, colchunk):
    c, hw = x.shape
    grid = hw // colchunk
    return pl.pallas_call(
        _tpose_body,
        grid=(grid,),
        in_specs=[pl.BlockSpec((c, colchunk), lambda g: (0, g))],
        out_specs=pl.BlockSpec((colchunk, c), lambda g: (g, 0)),
        out_shape=jax.ShapeDtypeStruct((hw, c), x.dtype),
    )(x)


def _make_table(feats):
    parts = []
    for f in feats:
        ch, h, w = f.shape[1], f.shape[2], f.shape[3]
        flat = f.reshape(ch, h * w)
        parts.append(_tpose(flat, min(2048, h * w)))
    return jnp.concatenate(parts, axis=0)


def _sc_pool(table, aux,
             compiler_params=pltpu.CompilerParams(needs_layout_passes=False)):
    m = aux.shape[0]
    info = plsc.get_sparse_core_info()
    nw = info.num_cores * info.num_subcores
    bpw = (m + nw - 1) // nw
    mesh = plsc.VectorSubcoreMesh(core_axis_name="c", subcore_axis_name="s")

    @functools.partial(
        pl.kernel,
        mesh=mesh,
        compiler_params=compiler_params,
        out_type=jax.ShapeDtypeStruct((m, OBOX), jnp.float32),
        scratch_types=[
            pltpu.VMEM((AUXF,), jnp.int32),
            pltpu.VMEM((C0ROWS, C), jnp.float32),
            pltpu.VMEM((NGRP * GROWS, C), jnp.float32),
            pltpu.VMEM((OBOX,), jnp.float32),
        ] + [pltpu.SemaphoreType.DMA] * (1 + NGRP),
    )
    def k(table_h, aux_h, out_h, aux_v, c0, rows, ob, sem0, *gsems):
        wid = lax.axis_index("s") * info.num_cores + lax.axis_index("c")
        start = wid * bpw
        cnt = jnp.maximum(jnp.minimum(bpw, m - start), 0)
        ii = lax.broadcasted_iota(jnp.int32, (NLANE,), 0) * NBIN

        def do_bin(buf, rbase, bin_, carry):
            accs = [jnp.zeros((NLANE,), jnp.float32) for _ in range(NC8)]
            wv = plsc.bitcast(aux_v[pl.ds(WOFF + bin_ * NTERM, NTERM)],
                              jnp.float32)
            for kk in range(NTERM):
                ws = wv[kk]
                for c8 in range(NC8):
                    accs[c8] = accs[c8] + ws * buf[
                        rbase + kk, pl.ds(c8 * NLANE, NLANE)]
            for c8 in range(NC8):
                plsc.store_scatter(
                    ob, [ii + (c8 * NLANE * NBIN + bin_)], accs[c8])
            return carry

        def box_body(b, carry):
            mm = start + b
            pltpu.sync_copy(aux_h.at[mm], aux_v)
            cp0 = pltpu.async_copy(
                table_h.at[aux_v.at[pl.ds(0, C0ROWS)]], c0, sem0)
            gcps = [
                pltpu.async_copy(
                    table_h.at[aux_v.at[pl.ds(C0ROWS + g * GROWS, GROWS)]],
                    rows.at[pl.ds(g * GROWS, GROWS)], gsems[g])
                for g in range(NGRP)
            ]
            cp0.wait()
            lax.fori_loop(
                0, OUT,
                lambda bb, cr: do_bin(c0, bb * NTERM, bb, cr), 0)
            for g in range(NGRP):
                gcps[g].wait()
                lax.fori_loop(
                    0, 2 * OUT,
                    lambda bb, cr, g=g: do_bin(
                        rows, g * GROWS + bb * NTERM, OUT + g * 2 * OUT + bb,
                        cr), 0)
            pltpu.sync_copy(ob, out_h.at[mm])
            return carry

        lax.fori_loop(0, cnt, box_body, 0)

    return k(table, aux)


def kernel(feat0, feat1, feat2, feat3, boxes1, boxes2):
    m = boxes1.shape[0]
    table = _make_table((feat0, feat1, feat2, feat3))
    aux = _prep(boxes1, boxes2)
    return _sc_pool(table, aux).reshape(m, C, OUT, OUT)


# speedup vs baseline: 1.0893x; 1.0893x over previous
import functools

import jax
import jax.numpy as jnp
from jax import lax
from jax.experimental import pallas as pl
from jax.experimental.pallas import tpu as pltpu
from jax.experimental.pallas import tpu_sc as plsc

OUT = 7
SR = 2
NBIN = OUT * OUT
NTERM = SR * SR * 4
P = NBIN * NTERM
C = 128
NLANE = 16
NC8 = C // NLANE
AUXF = 1792
WOFF = 896
C0ROWS = NTERM * OUT
GROWS = 2 * C0ROWS
NGRP = 3
SIZES = (128, 64, 32, 16)
BASES = (0.0, 16384.0, 20480.0, 21504.0)
NROWS = 21760
OBOX = C * NBIN


def _prep_body(b1_ref, b2_ref, aux_ref):
    b1 = b1_ref[...]
    b2 = b2_ref[...]
    n = b1.shape[0]

    ux1 = jnp.minimum(b1[:, 0:1], b2[:, 0:1])
    uy1 = jnp.minimum(b1[:, 1:2], b2[:, 1:2])
    ux2 = jnp.maximum(b1[:, 2:3], b2[:, 2:3])
    uy2 = jnp.maximum(b1[:, 3:4], b2[:, 3:4])

    s1 = jnp.sqrt((b1[:, 2:3] - b1[:, 0:1]) * (b1[:, 3:4] - b1[:, 1:2]))
    s2 = jnp.sqrt((b2[:, 2:3] - b2[:, 0:1]) * (b2[:, 3:4] - b2[:, 1:2]))
    s = jnp.minimum(s1, s2)
    lvl = jnp.clip(jnp.floor(4.0 + jnp.log2(s / 224.0 + 1e-6)), 2.0, 5.0) - 2.0

    scale = jnp.where(lvl == 0.0, 0.25,
            jnp.where(lvl == 1.0, 0.125,
            jnp.where(lvl == 2.0, 0.0625, 0.03125))).astype(jnp.float32)
    hf = 512.0 * scale
    base = jnp.where(lvl == 0.0, BASES[0],
           jnp.where(lvl == 1.0, BASES[1],
           jnp.where(lvl == 2.0, BASES[2], BASES[3]))).astype(jnp.float32)

    x1s = ux1 * scale
    y1s = uy1 * scale
    x2s = ux2 * scale
    y2s = uy2 * scale
    rw = jnp.maximum(x2s - x1s, 1.0)
    rh = jnp.maximum(y2s - y1s, 1.0)
    bw = rw / OUT
    bh = rh / OUT

    ff = lax.broadcasted_iota(jnp.int32, (n, WOFF), 1)
    in_p = ff < P
    pf = jnp.minimum(ff, P - 1).astype(jnp.float32)

    binf = jnp.floor(pf * 0.0625)
    kf = pf - 16.0 * binf
    i_f = jnp.floor(binf / 7.0)
    j_f = binf - 7.0 * i_f
    spf = jnp.floor(kf * 0.25)
    crf = kf - 4.0 * spf
    s_f = jnp.floor(spf * 0.5)
    t_f = spf - 2.0 * s_f
    a_f = jnp.floor(crf * 0.5)
    b_f = crf - 2.0 * a_f

    gy = y1s + (i_f + (s_f + 0.5) * 0.5) * bh
    gx = x1s + (j_f + (t_f + 0.5) * 0.5) * bw
    valid = ((gy > -1.0) & (gy < hf) & (gx > -1.0) & (gx < hf))
    y = jnp.clip(gy, 0.0, hf - 1.0)
    x = jnp.clip(gx, 0.0, hf - 1.0)
    y0f = jnp.floor(y)
    x0f = jnp.floor(x)
    y1f = jnp.minimum(y0f + 1.0, hf - 1.0)
    x1f = jnp.minimum(x0f + 1.0, hf - 1.0)
    ly = y - y0f
    lx = x - x0f
    wy = jnp.where(a_f == 0.0, 1.0 - ly, ly)
    wx = jnp.where(b_f == 0.0, 1.0 - lx, lx)
    wgt = wy * wx * 0.25 * valid.astype(jnp.float32)
    ysel = jnp.where(a_f == 0.0, y0f, y1f)
    xsel = jnp.where(b_f == 0.0, x0f, x1f)
    rowf = base + ysel * hf + xsel

    aux_ref[:, 0:WOFF] = jnp.where(in_p, rowf, 0.0).astype(jnp.int32)
    aux_ref[:, WOFF:AUXF] = jnp.where(
        in_p, jax.lax.bitcast_convert_type(wgt, jnp.int32), 0)


def _prep(boxes1, boxes2):
    m = boxes1.shape[0]
    chunk = 200 if m % 200 == 0 else m
    grid = m // chunk
    return pl.pallas_call(
        _prep_body,
        grid=(grid,),
        in_specs=[
            pl.BlockSpec((chunk, 4), lambda g: (g, 0)),
            pl.BlockSpec((chunk, 4), lambda g: (g, 0)),
        ],
        out_specs=pl.BlockSpec((chunk, AUXF), lambda g: (g, 0)),
        out_shape=jax.ShapeDtypeStruct((m, AUXF), jnp.int32),
    )(boxes1, boxes2)


def _tpose_body(x_ref, o_ref):
    o_ref[...] = x_ref[...].T


def _tpose(x, colchunk):
    c, hw = x.shape
    grid = hw // colchunk
    return pl.pallas_call(
        _tpose_body,
        grid=(grid,),
        in_specs=[pl.BlockSpec((c, colchunk), lambda g: (0, g))],
        out_specs=pl.BlockSpec((colchunk, c), lambda g: (g, 0)),
        out_shape=jax.ShapeDtypeStruct((hw, c), x.dtype),
    )(x)


def _make_table(feats):
    parts = []
    for f in feats:
        ch, h, w = f.shape[1], f.shape[2], f.shape[3]
        flat = f.reshape(ch, h * w)
        parts.append(_tpose(flat, min(2048, h * w)))
    return jnp.concatenate(parts, axis=0)


def _sc_pool(table, aux,
             compiler_params=pltpu.CompilerParams(needs_layout_passes=False)):
    m = aux.shape[0]
    info = plsc.get_sparse_core_info()
    nw = info.num_cores * info.num_subcores
    bpw = (m + nw - 1) // nw
    mesh = plsc.VectorSubcoreMesh(core_axis_name="c", subcore_axis_name="s")

    @functools.partial(
        pl.kernel,
        mesh=mesh,
        compiler_params=compiler_params,
        out_type=jax.ShapeDtypeStruct((m, NBIN, C), jnp.float32),
        scratch_types=[
            pltpu.VMEM((AUXF,), jnp.int32),
            pltpu.VMEM((C0ROWS, C), jnp.float32),
            pltpu.VMEM((NGRP * GROWS, C), jnp.float32),
            pltpu.VMEM((NBIN, C), jnp.float32),
        ] + [pltpu.SemaphoreType.DMA] * (1 + NGRP),
    )
    def k(table_h, aux_h, out_h, aux_v, c0, rows, ob, sem0, *gsems):
        wid = lax.axis_index("s") * info.num_cores + lax.axis_index("c")
        start = wid * bpw
        cnt = jnp.maximum(jnp.minimum(bpw, m - start), 0)

        def do_bin(buf, rbase, bin_, carry):
            accs = [jnp.zeros((NLANE,), jnp.float32) for _ in range(NC8)]
            wv = plsc.bitcast(aux_v[pl.ds(WOFF + bin_ * NTERM, NTERM)],
                              jnp.float32)
            for kk in range(NTERM):
                ws = wv[kk]
                for c8 in range(NC8):
                    accs[c8] = accs[c8] + ws * buf[
                        rbase + kk, pl.ds(c8 * NLANE, NLANE)]
            for c8 in range(NC8):
                ob[bin_, pl.ds(c8 * NLANE, NLANE)] = accs[c8]
            return carry

        def box_body(b, carry):
            mm = start + b
            pltpu.sync_copy(aux_h.at[mm], aux_v)
            cp0 = pltpu.async_copy(
                table_h.at[aux_v.at[pl.ds(0, C0ROWS)]], c0, sem0)
            gcps = [
                pltpu.async_copy(
                    table_h.at[aux_v.at[pl.ds(C0ROWS + g * GROWS, GROWS)]],
                    rows.at[pl.ds(g * GROWS, GROWS)], gsems[g])
                for g in range(NGRP)
            ]
            cp0.wait()
            lax.fori_loop(
                0, OUT,
                lambda bb, cr: do_bin(c0, bb * NTERM, bb, cr), 0)
            for g in range(NGRP):
                gcps[g].wait()
                lax.fori_loop(
                    0, 2 * OUT,
                    lambda bb, cr, g=g: do_bin(
                        rows, g * GROWS + bb * NTERM, OUT + g * 2 * OUT + bb,
                        cr), 0)
            pltpu.sync_copy(ob, out_h.at[mm])
            return carry

        lax.fori_loop(0, cnt, box_body, 0)

    return k(table, aux)


def kernel(feat0, feat1, feat2, feat3, boxes1, boxes2):
    m = boxes1.shape[0]
    table = _make_table((feat0, feat1, feat2, feat3))
    aux = _prep(boxes1, boxes2)
    out = _sc_pool(table, aux)
    return jnp.transpose(out, (0, 2, 1)).reshape(m, C, OUT, OUT)


# speedup vs baseline: 1.1109x; 1.0198x over previous
import functools

import jax
import jax.numpy as jnp
from jax import lax
from jax.experimental import pallas as pl
from jax.experimental.pallas import tpu as pltpu
from jax.experimental.pallas import tpu_sc as plsc

OUT = 7
SR = 2
NBIN = OUT * OUT
NTERM = SR * SR * 4
P = NBIN * NTERM
C = 128
NLANE = 16
NC8 = C // NLANE
AUXF = 1792
WOFF = 896
C0ROWS = NTERM * OUT
GROWS = 2 * C0ROWS
NGRP = 3
SIZES = (128, 64, 32, 16)
BASES = (0.0, 16384.0, 20480.0, 21504.0)
NROWS = 21760
OBOX = C * NBIN


def _prep_body(b1_ref, b2_ref, aux_ref):
    b1 = b1_ref[...]
    b2 = b2_ref[...]
    n = b1.shape[0]

    ux1 = jnp.minimum(b1[:, 0:1], b2[:, 0:1])
    uy1 = jnp.minimum(b1[:, 1:2], b2[:, 1:2])
    ux2 = jnp.maximum(b1[:, 2:3], b2[:, 2:3])
    uy2 = jnp.maximum(b1[:, 3:4], b2[:, 3:4])

    s1 = jnp.sqrt((b1[:, 2:3] - b1[:, 0:1]) * (b1[:, 3:4] - b1[:, 1:2]))
    s2 = jnp.sqrt((b2[:, 2:3] - b2[:, 0:1]) * (b2[:, 3:4] - b2[:, 1:2]))
    s = jnp.minimum(s1, s2)
    lvl = jnp.clip(jnp.floor(4.0 + jnp.log2(s / 224.0 + 1e-6)), 2.0, 5.0) - 2.0

    scale = jnp.where(lvl == 0.0, 0.25,
            jnp.where(lvl == 1.0, 0.125,
            jnp.where(lvl == 2.0, 0.0625, 0.03125))).astype(jnp.float32)
    hf = 512.0 * scale
    base = jnp.where(lvl == 0.0, BASES[0],
           jnp.where(lvl == 1.0, BASES[1],
           jnp.where(lvl == 2.0, BASES[2], BASES[3]))).astype(jnp.float32)

    x1s = ux1 * scale
    y1s = uy1 * scale
    x2s = ux2 * scale
    y2s = uy2 * scale
    rw = jnp.maximum(x2s - x1s, 1.0)
    rh = jnp.maximum(y2s - y1s, 1.0)
    bw = rw / OUT
    bh = rh / OUT

    ff = lax.broadcasted_iota(jnp.int32, (n, WOFF), 1)
    in_p = ff < P
    pf = jnp.minimum(ff, P - 1).astype(jnp.float32)

    binf = jnp.floor(pf * 0.0625)
    kf = pf - 16.0 * binf
    i_f = jnp.floor(binf / 7.0)
    j_f = binf - 7.0 * i_f
    spf = jnp.floor(kf * 0.25)
    crf = kf - 4.0 * spf
    s_f = jnp.floor(spf * 0.5)
    t_f = spf - 2.0 * s_f
    a_f = jnp.floor(crf * 0.5)
    b_f = crf - 2.0 * a_f

    gy = y1s + (i_f + (s_f + 0.5) * 0.5) * bh
    gx = x1s + (j_f + (t_f + 0.5) * 0.5) * bw
    valid = ((gy > -1.0) & (gy < hf) & (gx > -1.0) & (gx < hf))
    y = jnp.clip(gy, 0.0, hf - 1.0)
    x = jnp.clip(gx, 0.0, hf - 1.0)
    y0f = jnp.floor(y)
    x0f = jnp.floor(x)
    y1f = jnp.minimum(y0f + 1.0, hf - 1.0)
    x1f = jnp.minimum(x0f + 1.0, hf - 1.0)
    ly = y - y0f
    lx = x - x0f
    wy = jnp.where(a_f == 0.0, 1.0 - ly, ly)
    wx = jnp.where(b_f == 0.0, 1.0 - lx, lx)
    wgt = wy * wx * 0.25 * valid.astype(jnp.float32)
    ysel = jnp.where(a_f == 0.0, y0f, y1f)
    xsel = jnp.where(b_f == 0.0, x0f, x1f)
    rowf = base + ysel * hf + xsel

    aux_ref[:, 0:WOFF] = jnp.where(in_p, rowf, 0.0).astype(jnp.int32)
    aux_ref[:, WOFF:AUXF] = jnp.where(
        in_p, jax.lax.bitcast_convert_type(wgt, jnp.int32), 0)


def _prep(boxes1, boxes2):
    m = boxes1.shape[0]
    chunk = 200 if m % 200 == 0 else m
    grid = m // chunk
    return pl.pallas_call(
        _prep_body,
        grid=(grid,),
        in_specs=[
            pl.BlockSpec((chunk, 4), lambda g: (g, 0)),
            pl.BlockSpec((chunk, 4), lambda g: (g, 0)),
        ],
        out_specs=pl.BlockSpec((chunk, AUXF), lambda g: (g, 0)),
        out_shape=jax.ShapeDtypeStruct((m, AUXF), jnp.int32),
    )(boxes1, boxes2)


def _tpose_body(x_ref, o_ref):
    o_ref[...] = x_ref[...].T


def _tpose(x, colchunk):
    c, hw = x.shape
    grid = hw // colchunk
    return pl.pallas_call(
        _tpose_body,
        grid=(grid,),
        in_specs=[pl.BlockSpec((c, colchunk), lambda g: (0, g))],
        out_specs=pl.BlockSpec((colchunk, c), lambda g: (g, 0)),
        out_shape=jax.ShapeDtypeStruct((hw, c), x.dtype),
    )(x)


def _make_table(feats):
    parts = []
    for f in feats:
        ch, h, w = f.shape[1], f.shape[2], f.shape[3]
        flat = f.reshape(ch, h * w)
        parts.append(_tpose(flat, min(2048, h * w)))
    return jnp.concatenate(parts, axis=0)


def _sc_pool(table, aux,
             compiler_params=pltpu.CompilerParams(needs_layout_passes=False)):
    m = aux.shape[0]
    info = plsc.get_sparse_core_info()
    nw = info.num_cores * info.num_subcores
    bpw = (m + nw - 1) // nw
    mesh = plsc.VectorSubcoreMesh(core_axis_name="c", subcore_axis_name="s")

    @functools.partial(
        pl.kernel,
        mesh=mesh,
        compiler_params=compiler_params,
        out_type=jax.ShapeDtypeStruct((m, NBIN, C), jnp.float32),
        scratch_types=[
            pltpu.VMEM((AUXF,), jnp.int32),
            pltpu.VMEM((C0ROWS, C), jnp.float32),
            pltpu.VMEM((NGRP * GROWS, C), jnp.float32),
            pltpu.VMEM((NBIN, C), jnp.float32),
        ] + [pltpu.SemaphoreType.DMA] * (2 + NGRP),
    )
    def k(table_h, aux_h, out_h, aux_v, c0, rows, ob, sem0, semaux, *gsems):
        wid = lax.axis_index("s") * info.num_cores + lax.axis_index("c")
        start = wid * bpw
        cnt = jnp.maximum(jnp.minimum(bpw, m - start), 0)

        def do_bin(buf, rbase, bin_, carry):
            accs = [jnp.zeros((NLANE,), jnp.float32) for _ in range(NC8)]
            wv = plsc.bitcast(aux_v[pl.ds(WOFF + bin_ * NTERM, NTERM)],
                              jnp.float32)
            for kk in range(NTERM):
                ws = wv[kk]
                for c8 in range(NC8):
                    accs[c8] = accs[c8] + ws * buf[
                        rbase + kk, pl.ds(c8 * NLANE, NLANE)]
            for c8 in range(NC8):
                ob[bin_, pl.ds(c8 * NLANE, NLANE)] = accs[c8]
            return carry

        def box_body(b, carry):
            mm = start + b
            cp0 = pltpu.async_copy(
                table_h.at[aux_v.at[pl.ds(0, C0ROWS)]], c0, sem0)
            gcps = [
                pltpu.async_copy(
                    table_h.at[aux_v.at[pl.ds(C0ROWS + g * GROWS, GROWS)]],
                    rows.at[pl.ds(g * GROWS, GROWS)], gsems[g])
                for g in range(NGRP)
            ]
            cp0.wait()
            lax.fori_loop(
                0, OUT,
                lambda bb, cr: do_bin(c0, bb * NTERM, bb, cr), 0)
            for g in range(NGRP):
                gcps[g].wait()
                lax.fori_loop(
                    0, 2 * OUT,
                    lambda bb, cr, g=g: do_bin(
                        rows, g * GROWS + bb * NTERM, OUT + g * 2 * OUT + bb,
                        cr), 0)
            nxt = b + 1 < cnt

            @pl.when(nxt)
            def _():
                pltpu.async_copy(aux_h.at[jnp.minimum(mm + 1, m - 1)],
                                 aux_v, semaux)

            pltpu.sync_copy(ob, out_h.at[mm])

            @pl.when(nxt)
            def _():
                pltpu.make_async_copy(aux_h.at[0], aux_v, semaux).wait()
            return carry

        @pl.when(cnt > 0)
        def _():
            pltpu.sync_copy(aux_h.at[start], aux_v)

        lax.fori_loop(0, cnt, box_body, 0)

    return k(table, aux)


def kernel(feat0, feat1, feat2, feat3, boxes1, boxes2):
    m = boxes1.shape[0]
    table = _make_table((feat0, feat1, feat2, feat3))
    aux = _prep(boxes1, boxes2)
    out = _sc_pool(table, aux)
    return jnp.transpose(out, (0, 2, 1)).reshape(m, C, OUT, OUT)
